# baseline (speedup 1.0000x reference)
import functools

import jax
import jax.numpy as jnp
from jax import lax
from jax.experimental import pallas as pl
from jax.experimental.pallas import tpu as pltpu
from jax.experimental.pallas import tpu_sc as plsc

_N = 10000
_D = 128
_NC = 2
_NS = 16
_NW = _NC * _NS
_E_TOT = 320000
_CHUNK = 128
_NCHUNK = 80
_EPW = _NCHUNK * _CHUNK
_E_PAD = _EPW * _NW
_N_PAD = 10112
_RPS = _N_PAD // _NS
_PAGE = 20
_NPAGE = _NCHUNK // _PAGE
_N_PAD_DEG = 10240
_RPS_DEG = _N_PAD_DEG // _NS
_MROW = 2

_BLK = 1000
_GRID = _N // _BLK

_mesh = plsc.VectorSubcoreMesh(
    core_axis_name="c", subcore_axis_name="s", num_cores=_NC, num_subcores=_NS)


@functools.partial(
    pl.kernel,
    out_type=jax.ShapeDtypeStruct((_NC, _N_PAD_DEG), jnp.float32),
    mesh=_mesh,
    scratch_types=[
        pltpu.VMEM((_NCHUNK, _CHUNK), jnp.int32),
        pltpu.VMEM((_CHUNK,), jnp.float32),
        pltpu.VMEM_SHARED((_N_PAD_DEG,), jnp.float32),
    ],
)
def _sc_degree(dst_hbm, zeros1_hbm, out_hbm, didx, ones, dacc):
    c = lax.axis_index("c")
    s = lax.axis_index("s")
    wid = s * _NC + c
    pltpu.sync_copy(dst_hbm.at[wid], didx)
    for i in range(_CHUNK // 16):
        ones[pl.ds(i * 16, 16)] = jnp.ones((16,), jnp.float32)
    pltpu.sync_copy(zeros1_hbm.at[pl.ds(s * _RPS_DEG, _RPS_DEG)],
                    dacc.at[pl.ds(s * _RPS_DEG, _RPS_DEG)])
    plsc.subcore_barrier()

    def body(j, carry):
        pltpu.sync_copy(ones, dacc.at[didx.at[j]], add=True)
        return carry

    lax.fori_loop(0, _NCHUNK, body, 0)
    plsc.subcore_barrier()
    pltpu.sync_copy(dacc.at[pl.ds(s * _RPS_DEG, _RPS_DEG)],
                    out_hbm.at[c, pl.ds(s * _RPS_DEG, _RPS_DEG)])


@functools.partial(
    pl.kernel,
    out_type=jax.ShapeDtypeStruct((_NC, _N_PAD, _D), jnp.float32),
    mesh=_mesh,
    scratch_types=[
        pltpu.VMEM((_NCHUNK, _CHUNK), jnp.int32),
        pltpu.VMEM((_NCHUNK, _CHUNK), jnp.int32),
        pltpu.VMEM((_CHUNK, _D), jnp.float32),
        pltpu.SemaphoreType.DMA,
        pltpu.VMEM_SHARED((_N_PAD, _D), jnp.float32),
    ],
)
def _sc_message(src_hbm, dst_hbm, xs_hbm, zeros_hbm, out_hbm,
                sidx, didx, gbuf, sem, acc):
    c = lax.axis_index("c")
    s = lax.axis_index("s")
    wid = s * _NC + c
    pltpu.sync_copy(src_hbm.at[wid], sidx)
    pltpu.sync_copy(dst_hbm.at[wid], didx)
    pltpu.sync_copy(zeros_hbm.at[pl.ds(s * _RPS, _RPS)],
                    acc.at[pl.ds(s * _RPS, _RPS)])
    plsc.subcore_barrier()

    def body(j, carry):
        pltpu.async_copy(xs_hbm.at[sidx.at[j]], gbuf, sem).wait()
        pltpu.sync_copy(gbuf, acc.at[didx.at[j]], add=True)
        return carry

    lax.fori_loop(0, _NCHUNK, body, 0)
    plsc.subcore_barrier()
    pltpu.sync_copy(acc.at[pl.ds(s * _RPS, _RPS)],
                    out_hbm.at[c, pl.ds(s * _RPS, _RPS)])


def _type_of(i):
    return jnp.where(i < 4, 0, jnp.where(i < 7, 1, 2))


def _tc1_body(x_ref, w_ref, b_ref, d0_ref, d1_ref, x1_ref, xs_ref, rdeg_ref):
    deg = d0_ref[0, 0] + d1_ref[0, 0] + 1.0
    rdeg = lax.rsqrt(deg)
    rdeg_ref[0, 0] = rdeg
    x1 = jnp.dot(x_ref[...], w_ref[0], preferred_element_type=jnp.float32)
    x1 = x1 + b_ref[0]
    x1_ref[...] = x1
    xs_ref[...] = x1 * rdeg[:, None]


def _tc2_body(x1_ref, p0_ref, p1_ref, rdeg_ref, w_ref, b_ref, x2_ref, xs_ref):
    rdeg = rdeg_ref[0, 0][:, None]
    h = jnp.maximum(x1_ref[...] + rdeg * (p0_ref[...] + p1_ref[...]), 0.0)
    x2 = jnp.dot(h, w_ref[0], preferred_element_type=jnp.float32) + b_ref[0]
    x2_ref[...] = x2
    xs_ref[...] = x2 * rdeg


def _tc3_body(x2_ref, p0_ref, p1_ref, rdeg_ref, out_ref):
    rdeg = rdeg_ref[0, 0][:, None]
    out_ref[...] = x2_ref[...] + rdeg * (p0_ref[...] + p1_ref[...])


_row_spec = pl.BlockSpec((_BLK, _D), lambda i: (i, 0))
_w_spec = pl.BlockSpec((1, _D, _D), lambda i: (_type_of(i), 0, 0))
_b_spec = pl.BlockSpec((1, 1, _D), lambda i: (_type_of(i), 0, 0))
_vec_spec = pl.BlockSpec((1, 1, _BLK), lambda i: (i, 0, 0))

_tc1 = pl.pallas_call(
    _tc1_body,
    grid=(_GRID,),
    in_specs=[_row_spec, _w_spec, _b_spec, _vec_spec, _vec_spec],
    out_specs=[_row_spec, _row_spec, _vec_spec],
    out_shape=[
        jax.ShapeDtypeStruct((_N, _D), jnp.float32),
        jax.ShapeDtypeStruct((_N, _D), jnp.float32),
        jax.ShapeDtypeStruct((_GRID, 1, _BLK), jnp.float32),
    ],
)

_tc2 = pl.pallas_call(
    _tc2_body,
    grid=(_GRID,),
    in_specs=[_row_spec, _row_spec, _row_spec, _vec_spec, _w_spec, _b_spec],
    out_specs=[_row_spec, _row_spec],
    out_shape=[
        jax.ShapeDtypeStruct((_N, _D), jnp.float32),
        jax.ShapeDtypeStruct((_N, _D), jnp.float32),
    ],
)

_tc3 = pl.pallas_call(
    _tc3_body,
    grid=(_GRID,),
    in_specs=[_row_spec, _row_spec, _row_spec, _vec_spec],
    out_specs=_row_spec,
    out_shape=jax.ShapeDtypeStruct((_N, _D), jnp.float32),
)


def kernel(x0, x1, x2, edge_index0, edge_index1, edge_index2,
           W0_0, b0_0, W0_1, b0_1, W0_2, b0_2,
           W1_0, b1_0, W1_1, b1_1, W1_2, b1_2):
    x_cat = jnp.concatenate([x0, x1, x2], axis=0)
    W0 = jnp.stack([W0_0, W0_1, W0_2])
    b0 = jnp.stack([b0_0, b0_1, b0_2])[:, None, :]
    W1 = jnp.stack([W1_0, W1_1, W1_2])
    b1 = jnp.stack([b1_0, b1_1, b1_2])[:, None, :]

    src = jnp.concatenate([edge_index0[0], edge_index1[0], edge_index2[0]])
    dst = jnp.concatenate([edge_index0[1], edge_index1[1], edge_index2[1]])
    npad = _E_PAD - _E_TOT
    src_p = jnp.concatenate([src, jnp.zeros((npad,), jnp.int32)])
    dst_p = jnp.concatenate([dst, jnp.full((npad,), _N_PAD - 1, jnp.int32)])
    src_w = src_p.reshape(_NW, _NCHUNK, _CHUNK)
    dst_w = dst_p.reshape(_NW, _NCHUNK, _CHUNK)

    zeros1 = jnp.zeros((_N_PAD_DEG,), jnp.float32)
    zeros2 = jnp.zeros((_N_PAD, _D), jnp.float32)

    degp = _sc_degree(dst_w, zeros1)
    d0 = degp[0, :_N].reshape(_GRID, 1, _BLK)
    d1 = degp[1, :_N].reshape(_GRID, 1, _BLK)

    x1_full, xs1, rdeg_r = _tc1(x_cat, W0, b0, d0, d1)

    p = _sc_message(src_w, dst_w, xs1, zeros2)
    x2_full, xs2 = _tc2(x1_full, p[0, :_N], p[1, :_N], rdeg_r, W1, b1)

    q = _sc_message(src_w, dst_w, xs2, zeros2)
    return _tc3(x2_full, q[0, :_N], q[1, :_N], rdeg_r)


# speedup vs baseline: 2.7619x; 2.7619x over previous
import functools

import jax
import jax.numpy as jnp
from jax import lax
from jax.experimental import pallas as pl
from jax.experimental.pallas import tpu as pltpu
from jax.experimental.pallas import tpu_sc as plsc

_N = 10000
_D = 128
_NC = 2
_NS = 16
_NW = _NC * _NS
_E_TOT = 320000
_CHUNK = 128
_NCHUNK = 80
_EPW = _NCHUNK * _CHUNK
_E_PAD = _EPW * _NW
_N_PAD = 10240
_RPS = _N_PAD // _NS
_PAGE = 20
_NPAGE = _NCHUNK // _PAGE
_N_PAD_DEG = 10240
_RPS_DEG = _N_PAD_DEG // _NS
_MROW = 2

_BLK = 1000
_GRID = _N // _BLK

_mesh = plsc.VectorSubcoreMesh(
    core_axis_name="c", subcore_axis_name="s", num_cores=_NC, num_subcores=_NS)


@functools.partial(
    pl.kernel,
    out_type=jax.ShapeDtypeStruct((_NC, _N_PAD_DEG), jnp.float32),
    mesh=_mesh,
    scratch_types=[
        pltpu.VMEM((_NCHUNK, _CHUNK), jnp.int32),
        pltpu.VMEM((_CHUNK,), jnp.float32),
        pltpu.VMEM_SHARED((_N_PAD_DEG,), jnp.float32),
    ],
)
def _sc_degree(dst_hbm, zeros1_hbm, out_hbm, didx, ones, dacc):
    c = lax.axis_index("c")
    s = lax.axis_index("s")
    wid = s * _NC + c
    pltpu.sync_copy(dst_hbm.at[wid], didx)
    for i in range(_CHUNK // 16):
        ones[pl.ds(i * 16, 16)] = jnp.ones((16,), jnp.float32)
    pltpu.sync_copy(zeros1_hbm.at[pl.ds(s * _RPS_DEG, _RPS_DEG)],
                    dacc.at[pl.ds(s * _RPS_DEG, _RPS_DEG)])
    plsc.subcore_barrier()

    def body(j, carry):
        pltpu.sync_copy(ones, dacc.at[didx.at[j]], add=True)
        return carry

    lax.fori_loop(0, _NCHUNK, body, 0)
    plsc.subcore_barrier()
    pltpu.sync_copy(dacc.at[pl.ds(s * _RPS_DEG, _RPS_DEG)],
                    out_hbm.at[c, pl.ds(s * _RPS_DEG, _RPS_DEG)])


@functools.partial(
    pl.kernel,
    out_type=jax.ShapeDtypeStruct((_NC, _N_PAD, _D), jnp.float32),
    mesh=_mesh,
    scratch_types=[
        pltpu.VMEM((_NCHUNK, _CHUNK), jnp.int32),
        pltpu.VMEM((_NCHUNK, _CHUNK), jnp.int32),
        pltpu.VMEM((_CHUNK, _D), jnp.float32),
        pltpu.SemaphoreType.DMA,
        pltpu.VMEM_SHARED((_N_PAD, _D), jnp.float32),
    ],
)
def _sc_message(src_hbm, dst_hbm, xs_hbm, zeros_hbm, out_hbm,
                sidx, didx, gbuf, sem, acc):
    c = lax.axis_index("c")
    s = lax.axis_index("s")
    wid = s * _NC + c
    pltpu.sync_copy(src_hbm.at[wid], sidx)
    pltpu.sync_copy(dst_hbm.at[wid], didx)
    pltpu.sync_copy(zeros_hbm.at[pl.ds(s * _RPS, _RPS)],
                    acc.at[pl.ds(s * _RPS, _RPS)])
    plsc.subcore_barrier()

    def body(j, carry):
        pltpu.async_copy(xs_hbm.at[sidx.at[j]], gbuf, sem).wait()
        pltpu.sync_copy(gbuf, acc.at[didx.at[j]], add=True)
        return carry

    lax.fori_loop(0, _NCHUNK, body, 0)
    plsc.subcore_barrier()
    pltpu.sync_copy(acc.at[pl.ds(s * _RPS, _RPS)],
                    out_hbm.at[c, pl.ds(s * _RPS, _RPS)])


def _type_of(i):
    return jnp.where(i < 4, 0, jnp.where(i < 7, 1, 2))


def _tc1_body(x_ref, w_ref, b_ref, d0_ref, d1_ref, x1_ref, xs_ref, rdeg_ref):
    deg = d0_ref[0, 0] + d1_ref[0, 0] + 1.0
    rdeg = lax.rsqrt(deg)
    rdeg_ref[0, 0] = rdeg
    x1 = jnp.dot(x_ref[...], w_ref[0], preferred_element_type=jnp.float32)
    x1 = x1 + b_ref[0]
    x1_ref[...] = x1
    xs_ref[...] = x1 * rdeg[:, None]


def _tc2_body(x1_ref, p0_ref, p1_ref, rdeg_ref, w_ref, b_ref, x2_ref, xs_ref):
    rdeg = rdeg_ref[0, 0][:, None]
    agg = p0_ref[...] + p1_ref[...]
    h = jnp.maximum(x1_ref[...] + rdeg * agg, 0.0)
    x2 = jnp.dot(h, w_ref[0], preferred_element_type=jnp.float32) + b_ref[0]
    x2_ref[...] = x2
    xs_ref[...] = x2 * rdeg


def _tc3_body(x2_ref, p0_ref, p1_ref, rdeg_ref, out_ref):
    rdeg = rdeg_ref[0, 0][:, None]
    agg = p0_ref[...] + p1_ref[...]
    out_ref[...] = x2_ref[...] + rdeg * agg


_row_spec = pl.BlockSpec((_BLK, _D), lambda i: (i, 0))
_w_spec = pl.BlockSpec((1, _D, _D), lambda i: (_type_of(i), 0, 0))
_b_spec = pl.BlockSpec((1, 1, _D), lambda i: (_type_of(i), 0, 0))
_vec_spec = pl.BlockSpec((1, 1, _BLK), lambda i: (i, 0, 0))

_tc1 = pl.pallas_call(
    _tc1_body,
    grid=(_GRID,),
    in_specs=[_row_spec, _w_spec, _b_spec, _vec_spec, _vec_spec],
    out_specs=[_row_spec, _row_spec, _vec_spec],
    out_shape=[
        jax.ShapeDtypeStruct((_N, _D), jnp.float32),
        jax.ShapeDtypeStruct((_N, _D), jnp.float32),
        jax.ShapeDtypeStruct((_GRID, 1, _BLK), jnp.float32),
    ],
)

_tc2 = pl.pallas_call(
    _tc2_body,
    grid=(_GRID,),
    in_specs=[_row_spec, _row_spec, _row_spec, _vec_spec, _w_spec, _b_spec],
    out_specs=[_row_spec, _row_spec],
    out_shape=[
        jax.ShapeDtypeStruct((_N, _D), jnp.float32),
        jax.ShapeDtypeStruct((_N, _D), jnp.float32),
    ],
)

_tc3 = pl.pallas_call(
    _tc3_body,
    grid=(_GRID,),
    in_specs=[_row_spec, _row_spec, _row_spec, _vec_spec],
    out_specs=_row_spec,
    out_shape=jax.ShapeDtypeStruct((_N, _D), jnp.float32),
)


def kernel(x0, x1, x2, edge_index0, edge_index1, edge_index2,
           W0_0, b0_0, W0_1, b0_1, W0_2, b0_2,
           W1_0, b1_0, W1_1, b1_1, W1_2, b1_2):
    x_cat = jnp.concatenate([x0, x1, x2], axis=0)
    W0 = jnp.stack([W0_0, W0_1, W0_2])
    b0 = jnp.stack([b0_0, b0_1, b0_2])[:, None, :]
    W1 = jnp.stack([W1_0, W1_1, W1_2])
    b1 = jnp.stack([b1_0, b1_1, b1_2])[:, None, :]

    src = jnp.concatenate([edge_index0[0], edge_index1[0], edge_index2[0]])
    dst = jnp.concatenate([edge_index0[1], edge_index1[1], edge_index2[1]])
    npad = _E_PAD - _E_TOT
    pad_src = (jnp.arange(npad, dtype=jnp.int32) * 131) % _N
    pad_dst = _N + (jnp.arange(npad, dtype=jnp.int32) % (_N_PAD - _N))
    src_p = jnp.concatenate([src, pad_src])
    dst_p = jnp.concatenate([dst, pad_dst])
    src_w = src_p.reshape(_NW, _NCHUNK, _CHUNK)
    dst_w = dst_p.reshape(_NW, _NCHUNK, _CHUNK)

    zeros1 = jnp.zeros((_N_PAD_DEG,), jnp.float32)
    zeros2 = jnp.zeros((_N_PAD, _D), jnp.float32)

    degp = _sc_degree(dst_w, zeros1)
    d0 = degp[0, :_N].reshape(_GRID, 1, _BLK)
    d1 = degp[1, :_N].reshape(_GRID, 1, _BLK)

    x1_full, xs1, rdeg_r = _tc1(x_cat, W0, b0, d0, d1)

    p = _sc_message(src_w, dst_w, xs1, zeros2)
    x2_full, xs2 = _tc2(x1_full, p[0, :_N], p[1, :_N], rdeg_r, W1, b1)

    q = _sc_message(src_w, dst_w, xs2, zeros2)
    return _tc3(x2_full, q[0, :_N], q[1, :_N], rdeg_r)
